# Initial kernel scaffold; baseline (speedup 1.0000x reference)
import jax
import jax.numpy as jnp
from jax.experimental import pallas as pl


def kernel(features, edges, edges2, edge_features, W1_l, b1, W1_r, W2_l, b2, W2_r):
    raise NotImplementedError("write your pallas kernel here")


# speedup vs baseline: 3.3711x; 3.3711x over previous
import functools

import jax
import jax.numpy as jnp
from jax import lax
from jax.experimental import pallas as pl
from jax.experimental.pallas import tpu as pltpu
from jax.experimental.pallas import tpu_sc as plsc

N = 10000
E = 320000
D = 128
NC = 2
NS = 16
NW = NC * NS
N_PAD = 10240
RPS = N_PAD // NS
E_W = E // NW
CH = 128
NCHUNK = (E_W + CH - 1) // CH + 1
E_WP = NCHUNK * CH
HR = 8


def _sc_agg_body(with_cnt, *refs):
    if with_cnt:
        (x_hbm, src_hbm, dst_hbm, zs_hbm, zh_hbm, sum_out, cnt_out,
         src_v, dst_v, rows_v, hist_v, tmp_v, red_v, sum_sh, stage_sh,
         gsem) = refs
    else:
        (x_hbm, src_hbm, dst_hbm, zs_hbm, sum_out,
         src_v, dst_v, rows_v, sum_sh, gsem) = refs
    c = lax.axis_index("c")
    s = lax.axis_index("s")
    w = c * NS + s
    pltpu.sync_copy(zs_hbm, sum_sh.at[pl.ds(s * RPS, RPS)])
    if with_cnt:
        pltpu.sync_copy(zh_hbm, hist_v)
    plsc.subcore_barrier()
    ones16 = jnp.full((16,), 1.0, jnp.float32)

    def chunk(j, carry):
        pltpu.sync_copy(src_hbm.at[w, j], src_v)
        pltpu.sync_copy(dst_hbm.at[w, j], dst_v)
        gcp = pltpu.async_copy(x_hbm.at[src_v], rows_v, gsem)
        if with_cnt:
            def grp(g, cc):
                plsc.addupdate_scatter(
                    hist_v, [dst_v[pl.ds(g * 16, 16)]], ones16)
                return cc
            lax.fori_loop(0, CH // 16, grp, 0)
        gcp.wait()
        pltpu.sync_copy(rows_v, sum_sh.at[dst_v], add=True)
        return carry

    lax.fori_loop(0, NCHUNK, chunk, 0)
    if with_cnt:
        pltpu.sync_copy(hist_v, stage_sh.at[s])
    plsc.subcore_barrier()
    strip = pl.ds(s * RPS, RPS)
    if with_cnt:
        for r in range(NS // HR):
            pltpu.sync_copy(stage_sh.at[pl.ds(r * HR, HR), strip], tmp_v)

            def red(g, cc):
                acc = tmp_v[0, pl.ds(g * 16, 16)]
                for p in range(1, HR):
                    acc = acc + tmp_v[p, pl.ds(g * 16, 16)]
                if r == 0:
                    red_v[pl.ds(g * 16, 16)] = acc
                else:
                    red_v[pl.ds(g * 16, 16)] = red_v[pl.ds(g * 16, 16)] + acc
                return cc

            lax.fori_loop(0, RPS // 16, red, 0)
        pltpu.sync_copy(red_v, cnt_out.at[c, strip])
    pltpu.sync_copy(sum_sh.at[strip], sum_out.at[c, strip])


def _make_sc_agg(with_cnt):
    mesh = plsc.VectorSubcoreMesh(core_axis_name="c", subcore_axis_name="s")
    out_type = [jax.ShapeDtypeStruct((NC, N_PAD, D), jnp.float32)]
    scratch = [
        pltpu.VMEM((CH,), jnp.int32),
        pltpu.VMEM((CH,), jnp.int32),
        pltpu.VMEM((CH, D), jnp.float32),
    ]
    if with_cnt:
        out_type.append(jax.ShapeDtypeStruct((NC, N_PAD), jnp.float32))
        scratch += [
            pltpu.VMEM((N_PAD,), jnp.float32),
            pltpu.VMEM((HR, RPS), jnp.float32),
            pltpu.VMEM((RPS,), jnp.float32),
        ]
    scratch.append(pltpu.VMEM_SHARED((N_PAD, D), jnp.float32))
    if with_cnt:
        scratch.append(pltpu.VMEM_SHARED((NS, N_PAD), jnp.float32))
    scratch.append(pltpu.SemaphoreType.DMA)
    return pl.kernel(
        functools.partial(_sc_agg_body, with_cnt),
        out_type=tuple(out_type) if with_cnt else out_type[0],
        mesh=mesh,
        scratch_types=scratch,
        compiler_params=(
            pltpu.CompilerParams(needs_layout_passes=False)
            if with_cnt else None),
    )


def _tc_body(relu, s_ref, c_ref, x_ref, wl_ref, wr_ref, b_ref, o_ref):
    ssum = s_ref[0] + s_ref[1]
    cnt = c_ref[0] + c_ref[1]
    mean = ssum * (1.0 / jnp.maximum(cnt, 1.0))
    h = jnp.dot(mean, wl_ref[...], preferred_element_type=jnp.float32)
    h = h + jnp.dot(x_ref[...], wr_ref[...], preferred_element_type=jnp.float32)
    h = h + b_ref[...]
    if relu:
        h = jnp.maximum(h, 0.0)
    o_ref[...] = h


def _make_tc_layer(relu, block_rows=512):
    grid = (N_PAD // block_rows,)
    return pl.pallas_call(
        functools.partial(_tc_body, relu),
        grid=grid,
        in_specs=[
            pl.BlockSpec((NC, block_rows, D), lambda i: (0, i, 0)),
            pl.BlockSpec((NC, block_rows, 1), lambda i: (0, i, 0)),
            pl.BlockSpec((block_rows, D), lambda i: (i, 0)),
            pl.BlockSpec((D, D), lambda i: (0, 0)),
            pl.BlockSpec((D, D), lambda i: (0, 0)),
            pl.BlockSpec((1, D), lambda i: (0, 0)),
        ],
        out_specs=pl.BlockSpec((block_rows, D), lambda i: (i, 0)),
        out_shape=jax.ShapeDtypeStruct((N_PAD, D), jnp.float32),
    )


_sc_agg_cnt = _make_sc_agg(True)
_sc_agg = _make_sc_agg(False)
_tc_relu = _make_tc_layer(True)
_tc_lin = _make_tc_layer(False)


@jax.jit
def _run(features, edges, W1_l, b1, W1_r, W2_l, b2, W2_r):
    x = jnp.pad(features, ((0, N_PAD - N), (0, 0)))
    src = jnp.pad(edges[0].reshape(NW, E_W), ((0, 0), (0, E_WP - E_W)))
    dst = jnp.pad(edges[1].reshape(NW, E_W), ((0, 0), (0, E_WP - E_W)),
                  constant_values=N_PAD - 1)
    src = src.reshape(NW, NCHUNK, CH)
    dst = dst.reshape(NW, NCHUNK, CH)
    zs = jnp.zeros((RPS, D), jnp.float32)
    zh = jnp.zeros((N_PAD,), jnp.float32)

    sp1, cnt = _sc_agg_cnt(x, src, dst, zs, zh)
    cnt3 = cnt.reshape(NC, N_PAD, 1)
    x1 = _tc_relu(sp1, cnt3, x, W1_l.T, W1_r.T, b1.reshape(1, D))
    sp2 = _sc_agg(x1, src, dst, zs)
    out = _tc_lin(sp2, cnt3, x1, W2_l.T, W2_r.T, b2.reshape(1, D))
    return out[:N]


def kernel(features, edges, edges2, edge_features, W1_l, b1, W1_r, W2_l, b2, W2_r):
    return _run(features, edges, W1_l, b1, W1_r, W2_l, b2, W2_r)


# speedup vs baseline: 3.8554x; 1.1436x over previous
import functools

import jax
import jax.numpy as jnp
from jax import lax
from jax.experimental import pallas as pl
from jax.experimental.pallas import tpu as pltpu
from jax.experimental.pallas import tpu_sc as plsc

N = 10000
E = 320000
D = 128
NC = 2
NS = 16
NW = NC * NS
N_PAD = 10240
RPS = N_PAD // NS
E_W = E // NW
CH = 128
NCHUNK = (E_W + CH - 1) // CH + 1
E_WP = NCHUNK * CH
HR = 8

_MESH = plsc.VectorSubcoreMesh(core_axis_name="c", subcore_axis_name="s")


def _sc_agg_body(x_hbm, src_hbm, dst_hbm, zs_hbm, sum_out,
                 sva, svb, dva, dvb, rows_a, rows_b, sum_sh,
                 gsa, gsb, ssa, ssb, isa, isb):
    c = lax.axis_index("c")
    s = lax.axis_index("s")
    w = c * NS + s
    pltpu.sync_copy(zs_hbm, sum_sh.at[pl.ds(s * RPS, RPS)])
    pltpu.sync_copy(src_hbm.at[w, 0], sva)
    pltpu.sync_copy(dst_hbm.at[w, 0], dva)
    pltpu.sync_copy(src_hbm.at[w, 1], svb)
    pltpu.sync_copy(dst_hbm.at[w, 1], dvb)
    plsc.subcore_barrier()

    def body(i, carry):
        ga = pltpu.async_copy(x_hbm.at[sva], rows_a, gsa)
        gb = pltpu.async_copy(x_hbm.at[svb], rows_b, gsb)
        ga.wait()
        sa = pltpu.async_copy(rows_a, sum_sh.at[dva], ssa, add=True)
        pa1 = pltpu.async_copy(src_hbm.at[w, 2 * i + 2], sva, isa)
        gb.wait()
        sb = pltpu.async_copy(rows_b, sum_sh.at[dvb], ssb, add=True)
        pb1 = pltpu.async_copy(src_hbm.at[w, 2 * i + 3], svb, isb)
        sa.wait()
        pa2 = pltpu.async_copy(dst_hbm.at[w, 2 * i + 2], dva, isa)
        sb.wait()
        pb2 = pltpu.async_copy(dst_hbm.at[w, 2 * i + 3], dvb, isb)
        pa1.wait()
        pa2.wait()
        pb1.wait()
        pb2.wait()
        return carry

    lax.fori_loop(0, NCHUNK // 2, body, 0)
    plsc.subcore_barrier()
    strip = pl.ds(s * RPS, RPS)
    pltpu.sync_copy(sum_sh.at[strip], sum_out.at[c, strip])


_sc_agg = pl.kernel(
    _sc_agg_body,
    out_type=jax.ShapeDtypeStruct((NC, N_PAD, D), jnp.float32),
    mesh=_MESH,
    scratch_types=[
        pltpu.VMEM((CH,), jnp.int32),
        pltpu.VMEM((CH,), jnp.int32),
        pltpu.VMEM((CH,), jnp.int32),
        pltpu.VMEM((CH,), jnp.int32),
        pltpu.VMEM((CH, D), jnp.float32),
        pltpu.VMEM((CH, D), jnp.float32),
        pltpu.VMEM_SHARED((N_PAD, D), jnp.float32),
        pltpu.SemaphoreType.DMA,
        pltpu.SemaphoreType.DMA,
        pltpu.SemaphoreType.DMA,
        pltpu.SemaphoreType.DMA,
        pltpu.SemaphoreType.DMA,
        pltpu.SemaphoreType.DMA,
    ],
)


def _sc_cnt_body(dst_hbm, zh_hbm, cnt_out,
                      dall_v, hist_v, tmp_v, red_v, stage_sh):
    c = lax.axis_index("c")
    s = lax.axis_index("s")
    w = c * NS + s
    pltpu.sync_copy(zh_hbm, hist_v)
    pltpu.sync_copy(dst_hbm.at[w], dall_v)
    ones16 = jnp.full((16,), 1.0, jnp.float32)

    def grp(g, cc):
        plsc.addupdate_scatter(hist_v, [dall_v[pl.ds(g * 16, 16)]], ones16)
        return cc

    lax.fori_loop(0, E_WP // 16, grp, 0)
    pltpu.sync_copy(hist_v, stage_sh.at[s])
    plsc.subcore_barrier()
    strip = pl.ds(s * RPS, RPS)
    for r in range(NS // HR):
        pltpu.sync_copy(stage_sh.at[pl.ds(r * HR, HR), strip], tmp_v)

        def red(g, cc):
            acc = tmp_v[0, pl.ds(g * 16, 16)]
            for p in range(1, HR):
                acc = acc + tmp_v[p, pl.ds(g * 16, 16)]
            if r == 0:
                red_v[pl.ds(g * 16, 16)] = acc
            else:
                red_v[pl.ds(g * 16, 16)] = red_v[pl.ds(g * 16, 16)] + acc
            return cc

        lax.fori_loop(0, RPS // 16, red, 0)
    pltpu.sync_copy(red_v, cnt_out.at[c, strip])


_sc_cnt = pl.kernel(
    _sc_cnt_body,
    out_type=jax.ShapeDtypeStruct((NC, N_PAD), jnp.float32),
    mesh=_MESH,
    scratch_types=[
        pltpu.VMEM((E_WP,), jnp.int32),
        pltpu.VMEM((N_PAD,), jnp.float32),
        pltpu.VMEM((HR, RPS), jnp.float32),
        pltpu.VMEM((RPS,), jnp.float32),
        pltpu.VMEM_SHARED((NS, N_PAD), jnp.float32),
    ],
    compiler_params=pltpu.CompilerParams(needs_layout_passes=False),
)


def _tc_body(relu, s_ref, c_ref, x_ref, wl_ref, wr_ref, b_ref, o_ref):
    ssum = s_ref[0] + s_ref[1]
    cnt = c_ref[0] + c_ref[1]
    mean = ssum * (1.0 / jnp.maximum(cnt, 1.0))
    h = jnp.dot(mean, wl_ref[...], preferred_element_type=jnp.float32)
    h = h + jnp.dot(x_ref[...], wr_ref[...], preferred_element_type=jnp.float32)
    h = h + b_ref[...]
    if relu:
        h = jnp.maximum(h, 0.0)
    o_ref[...] = h


def _make_tc_layer(relu, block_rows=512):
    grid = (N_PAD // block_rows,)
    return pl.pallas_call(
        functools.partial(_tc_body, relu),
        grid=grid,
        in_specs=[
            pl.BlockSpec((NC, block_rows, D), lambda i: (0, i, 0)),
            pl.BlockSpec((NC, block_rows, 1), lambda i: (0, i, 0)),
            pl.BlockSpec((block_rows, D), lambda i: (i, 0)),
            pl.BlockSpec((D, D), lambda i: (0, 0)),
            pl.BlockSpec((D, D), lambda i: (0, 0)),
            pl.BlockSpec((1, D), lambda i: (0, 0)),
        ],
        out_specs=pl.BlockSpec((block_rows, D), lambda i: (i, 0)),
        out_shape=jax.ShapeDtypeStruct((N_PAD, D), jnp.float32),
    )


_tc_relu = _make_tc_layer(True)
_tc_lin = _make_tc_layer(False)


@jax.jit
def _run(features, edges, W1_l, b1, W1_r, W2_l, b2, W2_r):
    x = jnp.pad(features, ((0, N_PAD - N), (0, 0)))
    src = jnp.pad(edges[0].reshape(NW, E_W), ((0, 0), (0, E_WP - E_W)))
    dst = jnp.pad(edges[1].reshape(NW, E_W), ((0, 0), (0, E_WP - E_W)),
                  constant_values=N_PAD - 1)
    src3 = jnp.pad(src.reshape(NW, NCHUNK, CH), ((0, 0), (0, 2), (0, 0)))
    dst3 = jnp.pad(dst.reshape(NW, NCHUNK, CH), ((0, 0), (0, 2), (0, 0)))
    zs = jnp.zeros((RPS, D), jnp.float32)
    zh = jnp.zeros((N_PAD,), jnp.float32)

    cnt = _sc_cnt(dst, zh)
    cnt3 = cnt.reshape(NC, N_PAD, 1)
    sp1 = _sc_agg(x, src3, dst3, zs)
    x1 = _tc_relu(sp1, cnt3, x, W1_l.T, W1_r.T, b1.reshape(1, D))
    sp2 = _sc_agg(x1, src3, dst3, zs)
    out = _tc_lin(sp2, cnt3, x1, W2_l.T, W2_r.T, b2.reshape(1, D))
    return out[:N]


def kernel(features, edges, edges2, edge_features, W1_l, b1, W1_r, W2_l, b2, W2_r):
    return _run(features, edges, W1_l, b1, W1_r, W2_l, b2, W2_r)
